# baseline (speedup 1.0000x reference)
import functools

import jax
import jax.numpy as jnp
from jax import lax
from jax.experimental import pallas as pl
from jax.experimental.pallas import tpu as pltpu
from jax.experimental.pallas import tpu_sc as plsc

_NC, _NS, _L = 2, 16, 16
_NW = _NC * _NS

_V, _H, _R, _B = 100000, 26, 32, 16384
_BPW = _B // _NW
_C = 64
_NCHUNK = _BPW // _C
_ROWS = _C * _H
_IPG = 128
_NGATHER = _ROWS // _IPG
_IROWS_PER_W = _BPW * _H // _IPG


def _tree_prod(vs):
    while len(vs) > 1:
        nxt = [vs[i] * vs[i + 1] for i in range(0, len(vs) - 1, 2)]
        if len(vs) % 2:
            nxt.append(vs[-1])
        vs = nxt
    return vs[0]


@functools.partial(
    pl.kernel,
    out_type=jax.ShapeDtypeStruct((_B,), jnp.float32),
    mesh=plsc.VectorSubcoreMesh(core_axis_name="c", subcore_axis_name="s"),
    compiler_params=pltpu.CompilerParams(
        needs_layout_passes=False, use_tc_tiling_on_sc=False
    ),
    scratch_types=[
        pltpu.VMEM((_ROWS,), jnp.int32),
        pltpu.VMEM((_ROWS,), jnp.int32),
        pltpu.VMEM((_ROWS, _R), jnp.float32),
        pltpu.VMEM((_ROWS, _R), jnp.float32),
        pltpu.VMEM((_BPW,), jnp.float32),
        pltpu.VMEM((_C * _L,), jnp.float32),
        pltpu.VMEM((_R,), jnp.float32),
        pltpu.SemaphoreType.DMA,
        pltpu.SemaphoreType.DMA,
    ],
)
def _cp_kernel(table, idx2d, w, out, idx0, idx1, rows0, rows1, out_v, part_v,
               w_v, sem0, sem1):
    wid = lax.axis_index("s") * _NC + lax.axis_index("c")
    pltpu.sync_copy(w, w_v)

    idx_bufs = (idx0, idx1)
    row_bufs = (rows0, rows1)
    sems = (sem0, sem1)
    pending = [[], []]

    def fire(c, par):
        off = (wid * _BPW + c * _C) * _H
        pltpu.sync_copy(idx2d.at[pl.ds(off, _ROWS)], idx_bufs[par])
        cps = []
        for j in range(_NGATHER):
            cps.append(
                pltpu.async_copy(
                    table.at[idx_bufs[par].at[pl.ds(j * _IPG, _IPG)]],
                    row_bufs[par].at[pl.ds(j * _IPG, _IPG)],
                    sems[par],
                )
            )
        pending[par] = cps

    def compute(c, par):
        rows = row_bufs[par]
        iota = jax.lax.iota(jnp.int32, _L)
        w_lo = w_v[pl.ds(0, _L)]
        w_hi = w_v[pl.ds(_L, _L)]

        def b_body(b, carry):
            base = b * _H
            lo = [rows[base + m, pl.ds(0, _L)] for m in range(_H)]
            hi = [rows[base + m, pl.ds(_L, _L)] for m in range(_H)]
            v = _tree_prod(lo) * w_lo + _tree_prod(hi) * w_hi
            part_v[pl.ds(b * _L, _L)] = v
            return carry

        lax.fori_loop(0, _C, b_body, 0)

        def g_body(g, carry):
            pbase = (g * _L + iota) * _L
            acc = jnp.zeros((_L,), jnp.float32)
            for l in range(_L):
                acc = acc + plsc.load_gather(part_v, [pbase + l])
            out_v[pl.ds(c * _C + g * _L, _L)] = acc
            return carry

        lax.fori_loop(0, _C // _L, g_body, 0)

    fire(0, 0)
    for c in range(_NCHUNK):
        par = c % 2
        if c + 1 < _NCHUNK:
            fire(c + 1, 1 - par)
        for cp in pending[par]:
            cp.wait()
        compute(c, par)

    pltpu.sync_copy(out_v, out.at[pl.ds(wid * _BPW, _BPW)])


_VC = 12544
_NQ = 8


_MG = 4
_NG = (_H + _MG - 1) // _MG


def _transpose_body(x_ref, out_ref):
    z = x_ref[...]
    out_ref[...] = jnp.swapaxes(z.reshape(_MG * _R, _VC), 0, 1)


def _tc_relayout(tview):
    return pl.pallas_call(
        _transpose_body,
        grid=(_NG, _NQ),
        in_specs=[pl.BlockSpec((_MG, _R, _VC), lambda g, qc: (g, 0, qc))],
        out_specs=pl.BlockSpec(
            (_VC, _MG * _R), lambda g, qc: (g * _NQ + qc, 0)
        ),
        out_shape=jax.ShapeDtypeStruct(
            (_NG * _NQ * _VC, _MG * _R), jnp.float32
        ),
    )(tview)


def kernel(coords, factors, weights):
    tview = factors.transpose(0, 2, 1)
    packed = _tc_relayout(tview)
    table = packed.reshape(_NG * _NQ * _VC * _MG, _R)
    m = jnp.arange(_H, dtype=jnp.int32)[None, :]
    idx = (
        (m // _MG) * (_NQ * _VC * _MG)
        + (coords // _VC) * (_VC * _MG)
        + (coords % _VC) * _MG
        + (m % _MG)
    )
    return _cp_kernel(table, idx.reshape(_B * _H), weights)


# speedup vs baseline: 1.0248x; 1.0248x over previous
import functools

import jax
import jax.numpy as jnp
from jax import lax
from jax.experimental import pallas as pl
from jax.experimental.pallas import tpu as pltpu
from jax.experimental.pallas import tpu_sc as plsc

_NC, _NS, _L = 2, 16, 16
_NW = _NC * _NS

_V, _H, _R, _B = 100000, 26, 32, 16384
_BPW = _B // _NW
_C = 64
_NCHUNK = _BPW // _C
_ROWS = _C * _H
_IPG = 128
_NGATHER = _ROWS // _IPG
_IROWS_PER_W = _BPW * _H // _IPG


def _tree_prod(vs):
    while len(vs) > 1:
        nxt = [vs[i] * vs[i + 1] for i in range(0, len(vs) - 1, 2)]
        if len(vs) % 2:
            nxt.append(vs[-1])
        vs = nxt
    return vs[0]


@functools.partial(
    pl.kernel,
    out_type=jax.ShapeDtypeStruct((_B,), jnp.float32),
    mesh=plsc.VectorSubcoreMesh(core_axis_name="c", subcore_axis_name="s"),
    compiler_params=pltpu.CompilerParams(
        needs_layout_passes=False, use_tc_tiling_on_sc=False
    ),
    scratch_types=[
        pltpu.VMEM((_ROWS,), jnp.int32),
        pltpu.VMEM((_ROWS,), jnp.int32),
        pltpu.VMEM((_ROWS, _R), jnp.float32),
        pltpu.VMEM((_ROWS, _R), jnp.float32),
        pltpu.VMEM((_BPW,), jnp.float32),
        pltpu.VMEM((_C * _L,), jnp.float32),
        pltpu.VMEM((_R,), jnp.float32),
        pltpu.SemaphoreType.DMA,
        pltpu.SemaphoreType.DMA,
    ],
)
def _cp_kernel(table, idx2d, w, out, idx0, idx1, rows0, rows1, out_v, part_v,
               w_v, sem0, sem1):
    wid = lax.axis_index("s") * _NC + lax.axis_index("c")
    pltpu.sync_copy(w, w_v)

    idx_bufs = (idx0, idx1)
    row_bufs = (rows0, rows1)
    sems = (sem0, sem1)
    pending = [[], []]

    def fire(c, par):
        off = (wid * _BPW + c * _C) * _H
        pltpu.sync_copy(idx2d.at[pl.ds(off, _ROWS)], idx_bufs[par])
        cps = []
        for j in range(_NGATHER):
            cps.append(
                pltpu.async_copy(
                    table.at[idx_bufs[par].at[pl.ds(j * _IPG, _IPG)]],
                    row_bufs[par].at[pl.ds(j * _IPG, _IPG)],
                    sems[par],
                )
            )
        pending[par] = cps

    def compute(c, par):
        rows = row_bufs[par]
        iota = jax.lax.iota(jnp.int32, _L)
        w_lo = w_v[pl.ds(0, _L)]
        w_hi = w_v[pl.ds(_L, _L)]

        @plsc.parallel_loop(0, _C, unroll=2)
        def _(b):
            base = b * _H
            lo = [rows[base + m, pl.ds(0, _L)] for m in range(_H)]
            hi = [rows[base + m, pl.ds(_L, _L)] for m in range(_H)]
            v = _tree_prod(lo) * w_lo + _tree_prod(hi) * w_hi
            part_v[pl.ds(b * _L, _L)] = v

        @plsc.parallel_loop(0, _C // _L, unroll=1)
        def _(g):
            pbase = (g * _L + iota) * _L
            acc = jnp.zeros((_L,), jnp.float32)
            for l in range(_L):
                acc = acc + plsc.load_gather(part_v, [pbase + l])
            out_v[pl.ds(c * _C + g * _L, _L)] = acc

    fire(0, 0)
    for c in range(_NCHUNK):
        par = c % 2
        if c + 1 < _NCHUNK:
            fire(c + 1, 1 - par)
        for cp in pending[par]:
            cp.wait()
        compute(c, par)

    pltpu.sync_copy(out_v, out.at[pl.ds(wid * _BPW, _BPW)])


_VC = 25088
_NQ = 4


_MG = 4
_NG = (_H + _MG - 1) // _MG


def _transpose_body(x_ref, out_ref):
    z = x_ref[...]
    out_ref[...] = jnp.swapaxes(z.reshape(_MG * _R, _VC), 0, 1)


def _tc_relayout(tview):
    return pl.pallas_call(
        _transpose_body,
        grid=(_NG, _NQ),
        in_specs=[pl.BlockSpec((_MG, _R, _VC), lambda g, qc: (g, 0, qc))],
        out_specs=pl.BlockSpec(
            (_VC, _MG * _R), lambda g, qc: (g * _NQ + qc, 0)
        ),
        out_shape=jax.ShapeDtypeStruct(
            (_NG * _NQ * _VC, _MG * _R), jnp.float32
        ),
    )(tview)


def kernel(coords, factors, weights):
    tview = factors.transpose(0, 2, 1)
    packed = _tc_relayout(tview)
    table = packed.reshape(_NG * _NQ * _VC * _MG, _R)
    m = jnp.arange(_H, dtype=jnp.int32)[None, :]
    idx = (
        (m // _MG) * (_NQ * _VC * _MG)
        + (coords // _VC) * (_VC * _MG)
        + (coords % _VC) * _MG
        + (m % _MG)
    )
    return _cp_kernel(table, idx.reshape(_B * _H), weights)
